# Initial kernel scaffold; baseline (speedup 1.0000x reference)
import jax
import jax.numpy as jnp
from jax.experimental import pallas as pl


def kernel(node_feat, edge_index, rel_types, W_rel1, W_self1, a_src1, a_dst1, W_rel2, W_self2, a_src2, a_dst2):
    raise NotImplementedError("write your pallas kernel here")


# speedup vs baseline: 20.7163x; 20.7163x over previous
import functools

import jax
import jax.numpy as jnp
from jax import lax
from jax.experimental import pallas as pl
from jax.experimental.pallas import tpu as pltpu
from jax.experimental.pallas import tpu_sc as plsc

N = 10000
E = 320000
D = 128
R = 10

NC = 2
NS = 16
NW = NC * NS
EW = E // NW
C = 80
NCH = EW // C
STRIPE = 624
TAIL = N - NS * STRIPE
TAIL_OFF = NS * STRIPE

BN = 400


def _dense_body(h_ref, wall_ref, wself_ref, adst_ref, asrct_ref,
                hrel_ref, hself_ref, t_ref, s_ref):
    hb = h_ref[...]
    hr = jnp.dot(hb, wall_ref[...], preferred_element_type=jnp.float32)
    hrel_ref[...] = hr
    hs = jnp.dot(hb, wself_ref[...], preferred_element_type=jnp.float32)
    hself_ref[...] = hs
    t_ref[...] = jnp.sum(hs * adst_ref[...], axis=1, keepdims=True)
    prod = hr * asrct_ref[...]
    cols = [jnp.sum(prod[:, r * D:(r + 1) * D], axis=1, keepdims=True)
            for r in range(R)]
    s_ref[...] = jnp.concatenate(cols, axis=1)


def _dense(h, wall, wself, adst, asrct):
    grid = (N // BN,)
    return pl.pallas_call(
        _dense_body,
        grid=grid,
        in_specs=[
            pl.BlockSpec((BN, D), lambda i: (i, 0)),
            pl.BlockSpec((D, R * D), lambda i: (0, 0)),
            pl.BlockSpec((D, D), lambda i: (0, 0)),
            pl.BlockSpec((1, D), lambda i: (0, 0)),
            pl.BlockSpec((1, R * D), lambda i: (0, 0)),
        ],
        out_specs=[
            pl.BlockSpec((BN, R * D), lambda i: (i, 0)),
            pl.BlockSpec((BN, D), lambda i: (i, 0)),
            pl.BlockSpec((BN, 1), lambda i: (i, 0)),
            pl.BlockSpec((BN, R), lambda i: (i, 0)),
        ],
        out_shape=[
            jax.ShapeDtypeStruct((N, R * D), jnp.float32),
            jax.ShapeDtypeStruct((N, D), jnp.float32),
            jax.ShapeDtypeStruct((N, 1), jnp.float32),
            jax.ShapeDtypeStruct((N, R), jnp.float32),
        ],
    )(h, wall, wself, adst, asrct)


def _epi_body(a0_ref, a1_ref, d0_ref, d1_ref, hs_ref, o_ref, *, act):
    agg = a0_ref[...] + a1_ref[...]
    den = d0_ref[...] + d1_ref[...] + 1e-9
    x = agg / den + hs_ref[...]
    if act:
        x = jnp.where(x > 0, x, jnp.exp(jnp.minimum(x, 0.0)) - 1.0)
    o_ref[...] = x


def _epilogue(a0, a1, d0, d1, hself, act):
    grid = (N // BN,)
    return pl.pallas_call(
        functools.partial(_epi_body, act=act),
        grid=grid,
        in_specs=[
            pl.BlockSpec((BN, D), lambda i: (i, 0)),
            pl.BlockSpec((BN, D), lambda i: (i, 0)),
            pl.BlockSpec((BN, 1), lambda i: (i, 0)),
            pl.BlockSpec((BN, 1), lambda i: (i, 0)),
            pl.BlockSpec((BN, D), lambda i: (i, 0)),
        ],
        out_specs=pl.BlockSpec((BN, D), lambda i: (i, 0)),
        out_shape=jax.ShapeDtypeStruct((N, D), jnp.float32),
    )(a0, a1, d0, d1, hself)


def _edge_kernel_body(hrel2, sflat, t_hbm, srcrel_hbm, dst_hbm,
                      zrows,
                      agg_out, den_out,
                      agg_sh, den_sh,
                      t_loc, idx_sr, idx_d, sv, ex, rows,
                      sem0, sem1):
    c = lax.axis_index("c")
    sid = lax.axis_index("s")
    wid = sid * NC + c

    pltpu.sync_copy(t_hbm, t_loc)
    pltpu.sync_copy(zrows.at[pl.ds(sid * STRIPE, STRIPE)],
                    agg_sh.at[pl.ds(sid * STRIPE, STRIPE)])

    @pl.when(sid == 1)
    def _():
        for j0 in range(0, C, 16):
            sv[pl.ds(j0, 16)] = jnp.zeros((16,), jnp.float32)

        def zd(i, _):
            pltpu.sync_copy(sv, den_sh.at[pl.ds(i * C, C)])
            return 0

        lax.fori_loop(0, N // C, zd, 0)
        pltpu.sync_copy(zrows.at[pl.ds(TAIL_OFF, TAIL)],
                        agg_sh.at[pl.ds(TAIL_OFF, TAIL)])

    plsc.subcore_barrier()

    base0 = wid * EW

    def chunk(ci, _):
        base = base0 + ci * C
        pltpu.sync_copy(srcrel_hbm.at[pl.ds(base, C)], idx_sr)
        pltpu.sync_copy(dst_hbm.at[pl.ds(base, C)], idx_d)
        cp0 = pltpu.async_copy(hrel2.at[idx_sr], rows, sem0)
        cp1 = pltpu.async_copy(sflat.at[idx_sr], sv, sem1)
        cp0.wait()
        cp1.wait()

        for j0 in range(0, C, 16):
            d16 = idx_d[pl.ds(j0, 16)]
            x = sv[pl.ds(j0, 16)] + plsc.load_gather(t_loc, [d16])
            ex[pl.ds(j0, 16)] = jnp.exp(jnp.maximum(x, 0.2 * x))

        def scale(i16, _):
            j0 = i16 * 16
            ex16 = ex[pl.ds(j0, 16)]
            for jj in range(16):
                e = ex16[jj]
                for k in range(D // 16):
                    sl = pl.ds(k * 16, 16)
                    rows[j0 + jj, sl] = rows[j0 + jj, sl] * e
            return 0

        lax.fori_loop(0, C // 16, scale, 0)

        pltpu.sync_copy(rows, agg_sh.at[idx_d], add=True)
        pltpu.sync_copy(ex, den_sh.at[idx_d], add=True)
        return 0

    lax.fori_loop(0, NCH, chunk, 0)

    plsc.subcore_barrier()

    pltpu.sync_copy(agg_sh.at[pl.ds(sid * STRIPE, STRIPE)],
                    agg_out.at[pl.ds(c * N + sid * STRIPE, STRIPE)])

    @pl.when(sid == 0)
    def _():
        pltpu.sync_copy(agg_sh.at[pl.ds(TAIL_OFF, TAIL)],
                        agg_out.at[pl.ds(c * N + TAIL_OFF, TAIL)])
        pltpu.sync_copy(den_sh, t_loc)
        pltpu.sync_copy(t_loc, den_out.at[pl.ds(c * N, N)])


def _edge_phase(hrel2, sflat, t, srcrel, dst, zrows):
    mesh = plsc.VectorSubcoreMesh(core_axis_name="c", subcore_axis_name="s")
    fn = pl.kernel(
        _edge_kernel_body,
        out_type=[
            jax.ShapeDtypeStruct((NC * N, D), jnp.float32),
            jax.ShapeDtypeStruct((NC * N,), jnp.float32),
        ],
        mesh=mesh,
        compiler_params=pltpu.CompilerParams(needs_layout_passes=False),
        scratch_types=[
            pltpu.VMEM_SHARED((N, D), jnp.float32),
            pltpu.VMEM_SHARED((N,), jnp.float32),
            pltpu.VMEM((N,), jnp.float32),
            pltpu.VMEM((C,), jnp.int32),
            pltpu.VMEM((C,), jnp.int32),
            pltpu.VMEM((C,), jnp.float32),
            pltpu.VMEM((C,), jnp.float32),
            pltpu.VMEM((C, D), jnp.float32),
            pltpu.SemaphoreType.DMA,
            pltpu.SemaphoreType.DMA,
        ],
    )
    return fn(hrel2, sflat, t, srcrel, dst, zrows)


def _layer(h, srcrel, dst, wall, wself, adst, asrct, zrows, act):
    hrel, hself, t, s = _dense(h, wall, wself, adst, asrct)
    hrel2 = hrel.reshape(N * R, D)
    sflat = s.reshape(N * R)
    tvec = t.reshape(N)
    agg, den = _edge_phase(hrel2, sflat, tvec, srcrel, dst, zrows)
    a0, a1 = agg[:N], agg[N:]
    d0, d1 = den[:N, None], den[N:, None]
    return _epilogue(a0, a1, d0, d1, hself, act)


def kernel(node_feat, edge_index, rel_types, W_rel1, W_self1, a_src1, a_dst1,
           W_rel2, W_self2, a_src2, a_dst2):
    src = edge_index[0].astype(jnp.int32)
    dst = edge_index[1].astype(jnp.int32)
    rel = rel_types.astype(jnp.int32)
    srcrel = src * R + rel

    wall1 = W_rel1.transpose(1, 0, 2).reshape(D, R * D)
    wall2 = W_rel2.transpose(1, 0, 2).reshape(D, R * D)
    adst1 = a_dst1.reshape(1, D)
    adst2 = a_dst2.reshape(1, D)
    asrct1 = jnp.tile(a_src1, R).reshape(1, R * D)
    asrct2 = jnp.tile(a_src2, R).reshape(1, R * D)

    zrows = jnp.zeros((N, D), jnp.float32)

    h1 = _layer(node_feat, srcrel, dst, wall1, W_self1, adst1, asrct1,
                zrows, act=True)
    h2 = _layer(h1, srcrel, dst, wall2, W_self2, adst2, asrct2,
                zrows, act=False)
    return h2


# speedup vs baseline: 33.8382x; 1.6334x over previous
import functools

import jax
import jax.numpy as jnp
from jax import lax
from jax.experimental import pallas as pl
from jax.experimental.pallas import tpu as pltpu
from jax.experimental.pallas import tpu_sc as plsc

N = 10000
E = 320000
D = 128
R = 10

NC = 2
NS = 16
NW = NC * NS
EW = E // NW
C = 80
NCH = EW // C
BLK = 25
NBLK = NCH // BLK
STRIPE = 624
TAIL = N - NS * STRIPE
TAIL_OFF = NS * STRIPE

BN = 400


def _dense_body(h_ref, wall_ref, wself_ref, adst_ref, asrct_ref,
                hrel_ref, hself_ref, t_ref, s_ref):
    hb = h_ref[...]
    hr = jnp.dot(hb, wall_ref[...], preferred_element_type=jnp.float32)
    hrel_ref[...] = hr
    hs = jnp.dot(hb, wself_ref[...], preferred_element_type=jnp.float32)
    hself_ref[...] = hs
    t_ref[...] = jnp.sum(hs * adst_ref[...], axis=1, keepdims=True)
    prod = hr * asrct_ref[...]
    cols = [jnp.sum(prod[:, r * D:(r + 1) * D], axis=1, keepdims=True)
            for r in range(R)]
    s_ref[...] = jnp.concatenate(cols, axis=1)


def _dense(h, wall, wself, adst, asrct):
    grid = (N // BN,)
    return pl.pallas_call(
        _dense_body,
        grid=grid,
        in_specs=[
            pl.BlockSpec((BN, D), lambda i: (i, 0)),
            pl.BlockSpec((D, R * D), lambda i: (0, 0)),
            pl.BlockSpec((D, D), lambda i: (0, 0)),
            pl.BlockSpec((1, D), lambda i: (0, 0)),
            pl.BlockSpec((1, R * D), lambda i: (0, 0)),
        ],
        out_specs=[
            pl.BlockSpec((BN, R * D), lambda i: (i, 0)),
            pl.BlockSpec((BN, D), lambda i: (i, 0)),
            pl.BlockSpec((BN, 1), lambda i: (i, 0)),
            pl.BlockSpec((BN, R), lambda i: (i, 0)),
        ],
        out_shape=[
            jax.ShapeDtypeStruct((N, R * D), jnp.float32),
            jax.ShapeDtypeStruct((N, D), jnp.float32),
            jax.ShapeDtypeStruct((N, 1), jnp.float32),
            jax.ShapeDtypeStruct((N, R), jnp.float32),
        ],
    )(h, wall, wself, adst, asrct)


def _epi_body(a0_ref, a1_ref, d0_ref, d1_ref, hs_ref, o_ref, *, act):
    agg = a0_ref[...] + a1_ref[...]
    den = d0_ref[...] + d1_ref[...] + 1e-9
    x = agg / den + hs_ref[...]
    if act:
        x = jnp.where(x > 0, x, jnp.exp(jnp.minimum(x, 0.0)) - 1.0)
    o_ref[...] = x


def _epilogue(a0, a1, d0, d1, hself, act):
    grid = (N // BN,)
    return pl.pallas_call(
        functools.partial(_epi_body, act=act),
        grid=grid,
        in_specs=[
            pl.BlockSpec((BN, D), lambda i: (i, 0)),
            pl.BlockSpec((BN, D), lambda i: (i, 0)),
            pl.BlockSpec((BN, 1), lambda i: (i, 0)),
            pl.BlockSpec((BN, 1), lambda i: (i, 0)),
            pl.BlockSpec((BN, D), lambda i: (i, 0)),
        ],
        out_specs=pl.BlockSpec((BN, D), lambda i: (i, 0)),
        out_shape=jax.ShapeDtypeStruct((N, D), jnp.float32),
    )(a0, a1, d0, d1, hself)


def _edge_kernel_body(hrel2, sflat, t_hbm, srcrel_hbm, dst_hbm,
                      zrows,
                      agg_out, den_out,
                      agg_sh, den_sh,
                      t_loc, idx_sr, idx_d, sv2, ex, rows2,
                      sem_r, sem_v, sem_i):
    c = lax.axis_index("c")
    sid = lax.axis_index("s")
    wid = sid * NC + c

    pltpu.sync_copy(t_hbm, t_loc)
    pltpu.sync_copy(zrows.at[pl.ds(sid * STRIPE, STRIPE)],
                    agg_sh.at[pl.ds(sid * STRIPE, STRIPE)])

    @pl.when(sid == 1)
    def _():
        for j0 in range(0, C, 16):
            ex[pl.ds(j0, 16)] = jnp.zeros((16,), jnp.float32)

        def zd(i, _):
            pltpu.sync_copy(ex, den_sh.at[pl.ds(i * C, C)])
            return 0

        lax.fori_loop(0, N // C, zd, 0)
        pltpu.sync_copy(zrows.at[pl.ds(TAIL_OFF, TAIL)],
                        agg_sh.at[pl.ds(TAIL_OFF, TAIL)])

    plsc.subcore_barrier()

    for blk in range(NBLK):
        h0 = pltpu.async_copy(srcrel_hbm.at[wid, blk], idx_sr, sem_i)
        h1 = pltpu.async_copy(dst_hbm.at[wid, blk], idx_d, sem_i)
        h0.wait()
        h1.wait()

        p0 = (blk * BLK) % 2
        pltpu.async_copy(hrel2.at[idx_sr.at[0]],
                         rows2.at[pl.ds(p0 * C, C)], sem_r)
        pltpu.async_copy(sflat.at[idx_sr.at[0]],
                         sv2.at[pl.ds(p0 * C, C)], sem_v)

        def chunk(j, _):
            off = lax.rem(blk * BLK + j, 2) * C
            pltpu.make_async_copy(sflat.at[idx_sr.at[j]],
                                  sv2.at[pl.ds(off, C)], sem_v).wait()

            for j0 in range(0, C, 16):
                d16 = idx_d[j, pl.ds(j0, 16)]
                x = sv2[pl.ds(off + j0, 16)] + plsc.load_gather(t_loc, [d16])
                ex[pl.ds(j0, 16)] = jnp.exp(jnp.maximum(x, 0.2 * x))

            pltpu.make_async_copy(hrel2.at[idx_sr.at[j]],
                                  rows2.at[pl.ds(off, C)], sem_r).wait()

            @pl.when(j < BLK - 1)
            def _():
                noff = C - off
                pltpu.async_copy(hrel2.at[idx_sr.at[j + 1]],
                                 rows2.at[pl.ds(noff, C)], sem_r)
                pltpu.async_copy(sflat.at[idx_sr.at[j + 1]],
                                 sv2.at[pl.ds(noff, C)], sem_v)

            def scale(i16, _):
                j0 = i16 * 16
                ex16 = ex[pl.ds(j0, 16)]
                for jj in range(16):
                    e = ex16[jj]
                    for k in range(D // 16):
                        sl = pl.ds(k * 16, 16)
                        rows2[off + j0 + jj, sl] = rows2[off + j0 + jj, sl] * e
                return 0

            lax.fori_loop(0, C // 16, scale, 0)

            pltpu.sync_copy(rows2.at[pl.ds(off, C)],
                            agg_sh.at[idx_d.at[j]], add=True)
            pltpu.sync_copy(ex, den_sh.at[idx_d.at[j]], add=True)
            return 0

        lax.fori_loop(0, BLK, chunk, 0)

    plsc.subcore_barrier()

    pltpu.sync_copy(agg_sh.at[pl.ds(sid * STRIPE, STRIPE)],
                    agg_out.at[pl.ds(c * N + sid * STRIPE, STRIPE)])

    @pl.when(sid == 0)
    def _():
        pltpu.sync_copy(agg_sh.at[pl.ds(TAIL_OFF, TAIL)],
                        agg_out.at[pl.ds(c * N + TAIL_OFF, TAIL)])
        pltpu.sync_copy(den_sh, t_loc)
        pltpu.sync_copy(t_loc, den_out.at[pl.ds(c * N, N)])


def _edge_phase(hrel2, sflat, t, srcrel, dst, zrows):
    mesh = plsc.VectorSubcoreMesh(core_axis_name="c", subcore_axis_name="s")
    fn = pl.kernel(
        _edge_kernel_body,
        out_type=[
            jax.ShapeDtypeStruct((NC * N, D), jnp.float32),
            jax.ShapeDtypeStruct((NC * N,), jnp.float32),
        ],
        mesh=mesh,
        compiler_params=pltpu.CompilerParams(needs_layout_passes=False),
        scratch_types=[
            pltpu.VMEM_SHARED((N, D), jnp.float32),
            pltpu.VMEM_SHARED((N,), jnp.float32),
            pltpu.VMEM((N,), jnp.float32),
            pltpu.VMEM((BLK, C), jnp.int32),
            pltpu.VMEM((BLK, C), jnp.int32),
            pltpu.VMEM((2 * C,), jnp.float32),
            pltpu.VMEM((C,), jnp.float32),
            pltpu.VMEM((2 * C, D), jnp.float32),
            pltpu.SemaphoreType.DMA,
            pltpu.SemaphoreType.DMA,
            pltpu.SemaphoreType.DMA,
        ],
    )
    return fn(hrel2, sflat, t, srcrel, dst, zrows)


def _layer(h, srcrel, dst, wall, wself, adst, asrct, zrows, act):
    hrel, hself, t, s = _dense(h, wall, wself, adst, asrct)
    hrel2 = hrel.reshape(N * R, D)
    sflat = s.reshape(N * R)
    tvec = t.reshape(N)
    agg, den = _edge_phase(hrel2, sflat, tvec, srcrel, dst, zrows)
    a0, a1 = agg[:N], agg[N:]
    d0, d1 = den[:N, None], den[N:, None]
    return _epilogue(a0, a1, d0, d1, hself, act)


def kernel(node_feat, edge_index, rel_types, W_rel1, W_self1, a_src1, a_dst1,
           W_rel2, W_self2, a_src2, a_dst2):
    src = edge_index[0].astype(jnp.int32)
    dst = edge_index[1].astype(jnp.int32)
    rel = rel_types.astype(jnp.int32)
    srcrel = (src * R + rel).reshape(NW, NBLK, BLK, C)
    dst = dst.reshape(NW, NBLK, BLK, C)

    wall1 = W_rel1.transpose(1, 0, 2).reshape(D, R * D)
    wall2 = W_rel2.transpose(1, 0, 2).reshape(D, R * D)
    adst1 = a_dst1.reshape(1, D)
    adst2 = a_dst2.reshape(1, D)
    asrct1 = jnp.tile(a_src1, R).reshape(1, R * D)
    asrct2 = jnp.tile(a_src2, R).reshape(1, R * D)

    zrows = jnp.zeros((N, D), jnp.float32)

    h1 = _layer(node_feat, srcrel, dst, wall1, W_self1, adst1, asrct1,
                zrows, act=True)
    h2 = _layer(h1, srcrel, dst, wall2, W_self2, adst2, asrct2,
                zrows, act=False)
    return h2


# speedup vs baseline: 40.1034x; 1.1852x over previous
import functools

import jax
import jax.numpy as jnp
from jax import lax
from jax.experimental import pallas as pl
from jax.experimental.pallas import tpu as pltpu
from jax.experimental.pallas import tpu_sc as plsc

N = 10000
E = 320000
D = 128
R = 10

NC = 2
NS = 16
NW = NC * NS
EW = E // NW
C = 80
NCH = EW // C
BLK = 25
NBLK = NCH // BLK
STRIPE = 624
TAIL = N - NS * STRIPE
TAIL_OFF = NS * STRIPE

BN = 2000
NB = N // BN


def _dense_body(h_ref, wall_ref, wself_ref, adst_ref, asrc_ref,
                hrel_ref, hself_ref, t_ref, s_ref):
    i = pl.program_id(0)
    r = pl.program_id(1)
    hb = h_ref[...]
    hr = jnp.dot(hb, wall_ref[...], preferred_element_type=jnp.float32)
    hrel_ref[...] = hr
    s_ref[pl.ds(r * NB + i, 1), :] = jnp.sum(hr * asrc_ref[...],
                                             axis=1)[None, :]

    @pl.when(r == R - 1)
    def _():
        hs = jnp.dot(hb, wself_ref[...], preferred_element_type=jnp.float32)
        hself_ref[...] = hs
        t_ref[pl.ds(i, 1), :] = jnp.sum(hs * adst_ref[...], axis=1)[None, :]


def _dense(h, wall, wself, adst, asrc):
    return pl.pallas_call(
        _dense_body,
        grid=(NB, R),
        in_specs=[
            pl.BlockSpec((BN, D), lambda i, r: (i, 0)),
            pl.BlockSpec((D, D), lambda i, r: (0, r)),
            pl.BlockSpec((D, D), lambda i, r: (0, 0)),
            pl.BlockSpec((1, D), lambda i, r: (0, 0)),
            pl.BlockSpec((1, D), lambda i, r: (0, 0)),
        ],
        out_specs=[
            pl.BlockSpec((BN, D), lambda i, r: (r * NB + i, 0)),
            pl.BlockSpec((BN, D), lambda i, r: (i, 0)),
            pl.BlockSpec((NB, BN), lambda i, r: (0, 0)),
            pl.BlockSpec((R * NB, BN), lambda i, r: (0, 0)),
        ],
        out_shape=[
            jax.ShapeDtypeStruct((R * N, D), jnp.float32),
            jax.ShapeDtypeStruct((N, D), jnp.float32),
            jax.ShapeDtypeStruct((NB, BN), jnp.float32),
            jax.ShapeDtypeStruct((R * NB, BN), jnp.float32),
        ],
    )(h, wall, wself, adst, asrc)


def _epi_body(a0_ref, a1_ref, den_ref, hs_ref, o_ref, *, act):
    i = pl.program_id(0)
    agg = a0_ref[...] + a1_ref[...]
    den = (den_ref[i, :] + den_ref[NB + i, :] + 1e-9)[:, None]
    x = agg / den + hs_ref[...]
    if act:
        x = jnp.where(x > 0, x, jnp.exp(jnp.minimum(x, 0.0)) - 1.0)
    o_ref[...] = x


def _epilogue(agg, den2, hself, act):
    return pl.pallas_call(
        functools.partial(_epi_body, act=act),
        grid=(NB,),
        in_specs=[
            pl.BlockSpec((BN, D), lambda i: (i, 0)),
            pl.BlockSpec((BN, D), lambda i: (NB + i, 0)),
            pl.BlockSpec((2 * NB, BN), lambda i: (0, 0)),
            pl.BlockSpec((BN, D), lambda i: (i, 0)),
        ],
        out_specs=pl.BlockSpec((BN, D), lambda i: (i, 0)),
        out_shape=jax.ShapeDtypeStruct((N, D), jnp.float32),
    )(agg, agg, den2, hself)


def _edge_kernel_body(hrel2, sflat, t_hbm, srcrel_hbm, dst_hbm,
                      zrows,
                      agg_out, den_out,
                      agg_sh, den_sh,
                      t_loc, idx_sr, idx_d, sv2, ex, rows2,
                      sem_r, sem_v, sem_i):
    c = lax.axis_index("c")
    sid = lax.axis_index("s")
    wid = sid * NC + c

    pltpu.sync_copy(t_hbm, t_loc)
    pltpu.sync_copy(zrows.at[pl.ds(sid * STRIPE, STRIPE)],
                    agg_sh.at[pl.ds(sid * STRIPE, STRIPE)])

    @pl.when(sid == 1)
    def _():
        for j0 in range(0, C, 16):
            ex[pl.ds(j0, 16)] = jnp.zeros((16,), jnp.float32)

        def zd(i, _):
            pltpu.sync_copy(ex, den_sh.at[pl.ds(i * C, C)])
            return 0

        lax.fori_loop(0, N // C, zd, 0)
        pltpu.sync_copy(zrows.at[pl.ds(TAIL_OFF, TAIL)],
                        agg_sh.at[pl.ds(TAIL_OFF, TAIL)])

    plsc.subcore_barrier()

    for blk in range(NBLK):
        h0 = pltpu.async_copy(srcrel_hbm.at[wid, blk], idx_sr, sem_i)
        h1 = pltpu.async_copy(dst_hbm.at[wid, blk], idx_d, sem_i)
        h0.wait()
        h1.wait()

        p0 = (blk * BLK) % 2
        pltpu.async_copy(hrel2.at[idx_sr.at[0]],
                         rows2.at[pl.ds(p0 * C, C)], sem_r)
        pltpu.async_copy(sflat.at[idx_sr.at[0]],
                         sv2.at[pl.ds(p0 * C, C)], sem_v)

        def chunk(j, _):
            off = lax.rem(blk * BLK + j, 2) * C
            pltpu.make_async_copy(sflat.at[idx_sr.at[j]],
                                  sv2.at[pl.ds(off, C)], sem_v).wait()

            for j0 in range(0, C, 16):
                d16 = idx_d[j, pl.ds(j0, 16)]
                x = sv2[pl.ds(off + j0, 16)] + plsc.load_gather(t_loc, [d16])
                ex[pl.ds(j0, 16)] = jnp.exp(jnp.maximum(x, 0.2 * x))

            pltpu.make_async_copy(hrel2.at[idx_sr.at[j]],
                                  rows2.at[pl.ds(off, C)], sem_r).wait()

            @pl.when(j < BLK - 1)
            def _():
                noff = C - off
                pltpu.async_copy(hrel2.at[idx_sr.at[j + 1]],
                                 rows2.at[pl.ds(noff, C)], sem_r)
                pltpu.async_copy(sflat.at[idx_sr.at[j + 1]],
                                 sv2.at[pl.ds(noff, C)], sem_v)

            def scale(i16, _):
                j0 = i16 * 16
                ex16 = ex[pl.ds(j0, 16)]
                for jj in range(16):
                    e = ex16[jj]
                    for k in range(D // 16):
                        sl = pl.ds(k * 16, 16)
                        rows2[off + j0 + jj, sl] = rows2[off + j0 + jj, sl] * e
                return 0

            lax.fori_loop(0, C // 16, scale, 0)

            pltpu.sync_copy(rows2.at[pl.ds(off, C)],
                            agg_sh.at[idx_d.at[j]], add=True)
            pltpu.sync_copy(ex, den_sh.at[idx_d.at[j]], add=True)
            return 0

        lax.fori_loop(0, BLK, chunk, 0)

    plsc.subcore_barrier()

    pltpu.sync_copy(agg_sh.at[pl.ds(sid * STRIPE, STRIPE)],
                    agg_out.at[pl.ds(c * N + sid * STRIPE, STRIPE)])

    @pl.when(sid == 0)
    def _():
        pltpu.sync_copy(agg_sh.at[pl.ds(TAIL_OFF, TAIL)],
                        agg_out.at[pl.ds(c * N + TAIL_OFF, TAIL)])
        pltpu.sync_copy(den_sh, t_loc)
        pltpu.sync_copy(t_loc, den_out.at[pl.ds(c * N, N)])


def _edge_phase(hrel2, sflat, t, srcrel, dst, zrows):
    mesh = plsc.VectorSubcoreMesh(core_axis_name="c", subcore_axis_name="s")
    fn = pl.kernel(
        _edge_kernel_body,
        out_type=[
            jax.ShapeDtypeStruct((NC * N, D), jnp.float32),
            jax.ShapeDtypeStruct((NC * N,), jnp.float32),
        ],
        mesh=mesh,
        compiler_params=pltpu.CompilerParams(needs_layout_passes=False),
        scratch_types=[
            pltpu.VMEM_SHARED((N, D), jnp.float32),
            pltpu.VMEM_SHARED((N,), jnp.float32),
            pltpu.VMEM((N,), jnp.float32),
            pltpu.VMEM((BLK, C), jnp.int32),
            pltpu.VMEM((BLK, C), jnp.int32),
            pltpu.VMEM((2 * C,), jnp.float32),
            pltpu.VMEM((C,), jnp.float32),
            pltpu.VMEM((2 * C, D), jnp.float32),
            pltpu.SemaphoreType.DMA,
            pltpu.SemaphoreType.DMA,
            pltpu.SemaphoreType.DMA,
        ],
    )
    return fn(hrel2, sflat, t, srcrel, dst, zrows)


def _layer(h, srcrel, dst, wall, wself, adst, asrc, zrows, act):
    hrel2, hself, t2, s2 = _dense(h, wall, wself, adst, asrc)
    agg, den = _edge_phase(hrel2, s2.reshape(R * N), t2.reshape(N),
                           srcrel, dst, zrows)
    return _epilogue(agg, den.reshape(2 * NB, BN), hself, act)


def kernel(node_feat, edge_index, rel_types, W_rel1, W_self1, a_src1, a_dst1,
           W_rel2, W_self2, a_src2, a_dst2):
    src = edge_index[0].astype(jnp.int32)
    dst = edge_index[1].astype(jnp.int32)
    rel = rel_types.astype(jnp.int32)
    srcrel = (rel * N + src).reshape(NW, NBLK, BLK, C)
    dst = dst.reshape(NW, NBLK, BLK, C)

    wall1 = W_rel1.transpose(1, 0, 2).reshape(D, R * D)
    wall2 = W_rel2.transpose(1, 0, 2).reshape(D, R * D)
    adst1 = a_dst1.reshape(1, D)
    adst2 = a_dst2.reshape(1, D)
    asrc1 = a_src1.reshape(1, D)
    asrc2 = a_src2.reshape(1, D)

    zrows = jnp.zeros((N, D), jnp.float32)

    h1 = _layer(node_feat, srcrel, dst, wall1, W_self1, adst1, asrc1,
                zrows, act=True)
    h2 = _layer(h1, srcrel, dst, wall2, W_self2, adst2, asrc2,
                zrows, act=False)
    return h2
